# Initial kernel scaffold; baseline (speedup 1.0000x reference)
import jax
import jax.numpy as jnp
from jax.experimental import pallas as pl


def kernel(data_0, data_1, data_2, data_3, word_table, tag_table, is_content):
    raise NotImplementedError("write your pallas kernel here")


# speedup vs baseline: 3.3016x; 3.3016x over previous
import functools

import jax
import jax.numpy as jnp
from jax import lax
from jax.experimental import pallas as pl
from jax.experimental.pallas import tpu as pltpu
from jax.experimental.pallas import tpu_sc as plsc

_VOCAB = 100000
_DIM = 64
_B = 4096
_L = 200
_TAGS = 59
_TDIM = 4
_EDIM = _TDIM + 2
_EPAD = 8
_ODIM = _DIM + _EDIM
_EXT = _TAGS * 4

_N = _B * _L
_LANES = 128
_ROWS = _N // _LANES
_NC = 2
_NS = 16
_NW = _NC * _NS
_ROWS_PER_W = _ROWS // _NW
_CHUNK_ROWS = 4
_CHUNK = _CHUNK_ROWS * _LANES
_STEPS = _ROWS_PER_W // _CHUNK_ROWS


def _sc_gather(word_table, ext_table, idx0, eidx):
    mesh = plsc.VectorSubcoreMesh(core_axis_name="c", subcore_axis_name="s")

    @functools.partial(
        pl.kernel,
        mesh=mesh,
        compiler_params=pltpu.CompilerParams(use_tc_tiling_on_sc=False),
        out_type=jax.ShapeDtypeStruct((_N, _ODIM), jnp.float32),
        scratch_types=[
            pltpu.VMEM((_CHUNK_ROWS, _LANES), jnp.int32),
            pltpu.VMEM((_CHUNK_ROWS, _LANES), jnp.int32),
            pltpu.VMEM((_CHUNK, _DIM), jnp.float32),
            pltpu.VMEM((_CHUNK, _EPAD), jnp.float32),
            pltpu.SemaphoreType.DMA,
            pltpu.SemaphoreType.DMA,
        ],
    )
    def k(word_hbm, ext_hbm, idx0_hbm, eidx_hbm, out_hbm,
          i0_v, ie_v, wbuf, ebuf, gsem, ssem):
        wid = lax.axis_index("s") * _NC + lax.axis_index("c")
        row0 = wid * _ROWS_PER_W

        def body(j, carry):
            r = row0 + j * _CHUNK_ROWS
            pltpu.sync_copy(idx0_hbm.at[pl.ds(r, _CHUNK_ROWS)], i0_v)
            pltpu.sync_copy(eidx_hbm.at[pl.ds(r, _CHUNK_ROWS)], ie_v)
            gathers = []
            for b in range(_CHUNK_ROWS):
                gathers.append(pltpu.async_copy(
                    word_hbm.at[i0_v.at[b]],
                    wbuf.at[pl.ds(b * _LANES, _LANES)],
                    gsem))
                gathers.append(pltpu.async_copy(
                    ext_hbm.at[ie_v.at[b]],
                    ebuf.at[pl.ds(b * _LANES, _LANES)],
                    gsem))
            for g in gathers:
                g.wait()
            base = r * _LANES
            s0 = pltpu.async_copy(
                wbuf, out_hbm.at[pl.ds(base, _CHUNK), pl.ds(0, _DIM)], ssem)
            s1 = pltpu.async_copy(
                ebuf.at[:, pl.ds(0, _EDIM)],
                out_hbm.at[pl.ds(base, _CHUNK), pl.ds(_DIM, _EDIM)], ssem)
            s0.wait()
            s1.wait()
            return carry

        lax.fori_loop(0, _STEPS, body, 0)

    return k(word_table, ext_table, idx0, eidx)


def kernel(data_0, data_1, data_2, data_3, word_table, tag_table, is_content):
    s = jnp.asarray(is_content, jnp.float32)
    idx0 = jnp.transpose(data_0).reshape(_ROWS, _LANES)
    eidx = (jnp.transpose(data_1) * 4 + jnp.transpose(data_2) * 2
            + jnp.transpose(data_3)).reshape(_ROWS, _LANES)
    e = jnp.arange(_EXT, dtype=jnp.int32)
    ext = jnp.concatenate([
        jnp.repeat(tag_table, 4, axis=0),
        (((e >> 1) & 1).astype(jnp.float32) * s)[:, None],
        ((e & 1).astype(jnp.float32) * s)[:, None],
        jnp.zeros((_EXT, _EPAD - _EDIM), jnp.float32),
    ], axis=1)
    out = _sc_gather(word_table, ext, idx0, eidx)
    return out.reshape(_L, _B, _ODIM)
